# baseline (speedup 1.0000x reference)
import functools

import jax
import jax.numpy as jnp
from jax import lax
from jax.experimental import pallas as pl
from jax.experimental.pallas import tpu as pltpu
from jax.experimental.pallas import tpu_sc as plsc

B = 16384
D_GMF = 32
D_MLP = 64
NC = 2
NS = 16
NW = NC * NS
ROWS_PER_W = B // NW
CHUNK = 128
CHUNKS_PER_W = ROWS_PER_W // CHUNK


def _gather_body(uid_hbm, mid_hbm, ue_hbm, ie_hbm, um_hbm, im_hbm,
                 ue_out, ie_out, um_out, im_out,
                 idx_u, idx_m, ue_v, ie_v, um_v, im_v, sem):
    wid = lax.axis_index("s") * NC + lax.axis_index("c")
    base = wid * ROWS_PER_W
    pltpu.sync_copy(uid_hbm.at[pl.ds(wid * CHUNKS_PER_W, CHUNKS_PER_W)], idx_u)
    pltpu.sync_copy(mid_hbm.at[pl.ds(wid * CHUNKS_PER_W, CHUNKS_PER_W)], idx_m)
    copies = []
    for j in range(CHUNKS_PER_W):
        sl = pl.ds(j * CHUNK, CHUNK)
        copies.append(pltpu.async_copy(ue_hbm.at[idx_u.at[j]], ue_v.at[sl], sem))
        copies.append(pltpu.async_copy(ie_hbm.at[idx_m.at[j]], ie_v.at[sl], sem))
        copies.append(pltpu.async_copy(um_hbm.at[idx_u.at[j]], um_v.at[sl], sem))
        copies.append(pltpu.async_copy(im_hbm.at[idx_m.at[j]], im_v.at[sl], sem))
    for c in copies:
        c.wait()
    out_sl = pl.ds(base, ROWS_PER_W)
    pltpu.sync_copy(ue_v, ue_out.at[out_sl])
    pltpu.sync_copy(ie_v, ie_out.at[out_sl])
    pltpu.sync_copy(um_v, um_out.at[out_sl])
    pltpu.sync_copy(im_v, im_out.at[out_sl])


def _sc_gather(uid2d, mid2d, ue, ie, um, im):
    mesh = plsc.VectorSubcoreMesh(core_axis_name="c", subcore_axis_name="s")
    f = functools.partial(
        pl.kernel,
        mesh=mesh,
        out_type=[
            jax.ShapeDtypeStruct((B, D_GMF), jnp.float32),
            jax.ShapeDtypeStruct((B, D_GMF), jnp.float32),
            jax.ShapeDtypeStruct((B, D_MLP), jnp.float32),
            jax.ShapeDtypeStruct((B, D_MLP), jnp.float32),
        ],
        scratch_types=[
            pltpu.VMEM((CHUNKS_PER_W, CHUNK), jnp.int32),
            pltpu.VMEM((CHUNKS_PER_W, CHUNK), jnp.int32),
            pltpu.VMEM((ROWS_PER_W, D_GMF), jnp.float32),
            pltpu.VMEM((ROWS_PER_W, D_GMF), jnp.float32),
            pltpu.VMEM((ROWS_PER_W, D_MLP), jnp.float32),
            pltpu.VMEM((ROWS_PER_W, D_MLP), jnp.float32),
            pltpu.SemaphoreType.DMA,
        ],
        compiler_params=pltpu.CompilerParams(use_tc_tiling_on_sc=False),
    )(_gather_body)
    return f(uid2d, mid2d, ue, ie, um, im)


BLK = 2048


def _dense_body(ue_ref, ie_ref, um_ref, im_ref,
                W1_ref, b1_ref, W2_ref, b2_ref, Wo_ref, bo_ref, out_ref):
    gmf = ue_ref[...] * ie_ref[...]
    x = jnp.dot(um_ref[...], W1_ref[0:D_MLP, :],
                preferred_element_type=jnp.float32,
                precision=lax.Precision.HIGHEST)
    x = x + jnp.dot(im_ref[...], W1_ref[D_MLP:2 * D_MLP, :],
                    preferred_element_type=jnp.float32,
                    precision=lax.Precision.HIGHEST)
    x = jax.nn.relu(x + b1_ref[...])
    x = jnp.dot(x, W2_ref[...],
                preferred_element_type=jnp.float32,
                precision=lax.Precision.HIGHEST)
    x = jax.nn.relu(x + b2_ref[...])
    wg = Wo_ref[0:D_GMF, 0]
    wm = Wo_ref[D_GMF:2 * D_GMF, 0]
    logit = (jnp.sum(gmf * wg[None, :], axis=-1)
             + jnp.sum(x * wm[None, :], axis=-1)
             + bo_ref[...])
    out_ref[...] = jax.nn.sigmoid(logit)


def _tc_dense(ue_g, ie_g, um_g, im_g, W1, b1, W2, b2, Wo, bo):
    grid = (B // BLK,)
    blk = lambda d: pl.BlockSpec((BLK, d), lambda i: (i, 0))
    full = lambda s: pl.BlockSpec(s, lambda i: tuple(0 for _ in s))
    return pl.pallas_call(
        _dense_body,
        grid=grid,
        in_specs=[
            blk(D_GMF), blk(D_GMF), blk(D_MLP), blk(D_MLP),
            full(W1.shape), full(b1.shape), full(W2.shape), full(b2.shape),
            full(Wo.shape), full(bo.shape),
        ],
        out_specs=pl.BlockSpec((BLK,), lambda i: (i,)),
        out_shape=jax.ShapeDtypeStruct((B,), jnp.float32),
    )(ue_g, ie_g, um_g, im_g, W1, b1, W2, b2, Wo, bo)


def kernel(user_emb, item_emb, user_emb_mlp, item_emb_mlp,
           W1, b1, W2, b2, Wo, bo, user_ids, movie_ids):
    uid2d = user_ids.astype(jnp.int32).reshape(B // CHUNK, CHUNK)
    mid2d = movie_ids.astype(jnp.int32).reshape(B // CHUNK, CHUNK)
    ue_g, ie_g, um_g, im_g = _sc_gather(
        uid2d, mid2d, user_emb, item_emb, user_emb_mlp, item_emb_mlp)
    return _tc_dense(ue_g, ie_g, um_g, im_g, W1, b1, W2, b2, Wo, bo)


# speedup vs baseline: 1.2523x; 1.2523x over previous
import functools

import jax
import jax.numpy as jnp
from jax import lax
from jax.experimental import pallas as pl
from jax.experimental.pallas import tpu as pltpu
from jax.experimental.pallas import tpu_sc as plsc

B = 16384
D_GMF = 32
D_MLP = 64
NC = 2
NS = 16
NW = NC * NS
ROWS_PER_W = B // NW
CHUNK = 128
CHUNKS_PER_W = ROWS_PER_W // CHUNK


def _gather_side_body(ids_hbm, emb_hbm, mlp_hbm, out,
                      idx, emb_v, mlp_v, sem):
    wid = lax.axis_index("s") * NC + lax.axis_index("c")
    base = wid * ROWS_PER_W
    pltpu.sync_copy(ids_hbm.at[pl.ds(wid * CHUNKS_PER_W, CHUNKS_PER_W)], idx)
    copies = []
    for j in range(CHUNKS_PER_W):
        sl = pl.ds(j * CHUNK, CHUNK)
        copies.append(pltpu.async_copy(emb_hbm.at[idx.at[j]], emb_v.at[sl], sem))
        copies.append(pltpu.async_copy(mlp_hbm.at[idx.at[j]], mlp_v.at[sl], sem))
    for c in copies:
        c.wait()
    out_sl = pl.ds(base, ROWS_PER_W)
    pltpu.sync_copy(mlp_v, out.at[out_sl, pl.ds(0, D_MLP)])
    pltpu.sync_copy(emb_v, out.at[out_sl, pl.ds(D_MLP, D_GMF)])


def _sc_gather_side(ids2d, emb, mlp):
    mesh = plsc.VectorSubcoreMesh(core_axis_name="c", subcore_axis_name="s")
    f = functools.partial(
        pl.kernel,
        mesh=mesh,
        out_type=jax.ShapeDtypeStruct((B, 128), jnp.float32),
        scratch_types=[
            pltpu.VMEM((CHUNKS_PER_W, CHUNK), jnp.int32),
            pltpu.VMEM((ROWS_PER_W, D_GMF), jnp.float32),
            pltpu.VMEM((ROWS_PER_W, D_MLP), jnp.float32),
            pltpu.SemaphoreType.DMA,
        ],
        compiler_params=pltpu.CompilerParams(use_tc_tiling_on_sc=False),
    )(_gather_side_body)
    return f(ids2d, emb, mlp)


BLK = 2048


def _dense_body(u_ref, i_ref, W1_ref, b1_ref, W2_ref, b2_ref,
                Wo_ref, bo_ref, out_ref):
    u = u_ref[...]
    i = i_ref[...]
    gmf = (u[:, D_MLP:D_MLP + D_GMF] * i[:, D_MLP:D_MLP + D_GMF])
    x = jnp.dot(u[:, 0:D_MLP], W1_ref[0:D_MLP, :],
                preferred_element_type=jnp.float32)
    x = x + jnp.dot(i[:, 0:D_MLP], W1_ref[D_MLP:2 * D_MLP, :],
                    preferred_element_type=jnp.float32)
    x = jax.nn.relu(x + b1_ref[...])
    x = jnp.dot(x, W2_ref[...], preferred_element_type=jnp.float32)
    x = jax.nn.relu(x + b2_ref[...])
    wg = Wo_ref[0:D_GMF, 0]
    wm = Wo_ref[D_GMF:2 * D_GMF, 0]
    logit = (jnp.sum(gmf * wg[None, :], axis=-1)
             + jnp.sum(x * wm[None, :], axis=-1)
             + bo_ref[...])
    out_ref[...] = jax.nn.sigmoid(logit)


def _tc_dense(u_pack, i_pack, W1, b1, W2, b2, Wo, bo):
    grid = (B // BLK,)
    blk = pl.BlockSpec((BLK, 128), lambda i: (i, 0))
    full = lambda s: pl.BlockSpec(s, lambda i: tuple(0 for _ in s))
    return pl.pallas_call(
        _dense_body,
        grid=grid,
        in_specs=[
            blk, blk,
            full(W1.shape), full(b1.shape), full(W2.shape), full(b2.shape),
            full(Wo.shape), full(bo.shape),
        ],
        out_specs=pl.BlockSpec((BLK,), lambda i: (i,)),
        out_shape=jax.ShapeDtypeStruct((B,), jnp.float32),
    )(u_pack, i_pack, W1, b1, W2, b2, Wo, bo)


def kernel(user_emb, item_emb, user_emb_mlp, item_emb_mlp,
           W1, b1, W2, b2, Wo, bo, user_ids, movie_ids):
    uid2d = user_ids.astype(jnp.int32).reshape(B // CHUNK, CHUNK)
    mid2d = movie_ids.astype(jnp.int32).reshape(B // CHUNK, CHUNK)
    u_pack = _sc_gather_side(uid2d, user_emb, user_emb_mlp)
    i_pack = _sc_gather_side(mid2d, item_emb, item_emb_mlp)
    return _tc_dense(u_pack, i_pack, W1, b1, W2, b2, Wo, bo)


# speedup vs baseline: 1.2555x; 1.0025x over previous
import functools

import jax
import jax.numpy as jnp
from jax import lax
from jax.experimental import pallas as pl
from jax.experimental.pallas import tpu as pltpu
from jax.experimental.pallas import tpu_sc as plsc

B = 16384
D_GMF = 32
D_MLP = 64
NC = 2
NS = 16
NW = NC * NS
ROWS_PER_W = B // NW
CHUNK = 128
CHUNKS_PER_W = ROWS_PER_W // CHUNK


def _gather_side_body(ids_hbm, emb_hbm, mlp_hbm, out,
                      idx, emb_v, mlp_v, sem):
    wid = lax.axis_index("s") * NC + lax.axis_index("c")
    base = wid * ROWS_PER_W
    pltpu.sync_copy(ids_hbm.at[pl.ds(wid * CHUNKS_PER_W, CHUNKS_PER_W)], idx)
    copies = []
    for j in range(CHUNKS_PER_W):
        sl = pl.ds(j * CHUNK, CHUNK)
        copies.append(pltpu.async_copy(emb_hbm.at[idx.at[j]], emb_v.at[sl], sem))
        copies.append(pltpu.async_copy(mlp_hbm.at[idx.at[j]], mlp_v.at[sl], sem))
    for c in copies:
        c.wait()
    out_sl = pl.ds(base, ROWS_PER_W)
    pltpu.sync_copy(mlp_v, out.at[out_sl, pl.ds(0, D_MLP)])
    pltpu.sync_copy(emb_v, out.at[out_sl, pl.ds(D_MLP, D_GMF)])


def _sc_gather_side(ids2d, emb, mlp):
    mesh = plsc.VectorSubcoreMesh(core_axis_name="c", subcore_axis_name="s")
    f = functools.partial(
        pl.kernel,
        mesh=mesh,
        out_type=jax.ShapeDtypeStruct((B, 128), jnp.float32),
        scratch_types=[
            pltpu.VMEM((CHUNKS_PER_W, CHUNK), jnp.int32),
            pltpu.VMEM((ROWS_PER_W, D_GMF), jnp.float32),
            pltpu.VMEM((ROWS_PER_W, D_MLP), jnp.float32),
            pltpu.SemaphoreType.DMA,
        ],
        compiler_params=pltpu.CompilerParams(use_tc_tiling_on_sc=False),
    )(_gather_side_body)
    return f(ids2d, emb, mlp)


BLK = 2048


def _dense_body(u_ref, i_ref, W1_ref, b1_ref, W2_ref, b2_ref,
                Wo_ref, bo_ref, out_ref):
    u = u_ref[...].reshape(BLK, 128)
    i = i_ref[...].reshape(BLK, 128)
    gmf = (u[:, D_MLP:D_MLP + D_GMF] * i[:, D_MLP:D_MLP + D_GMF])
    x = jnp.dot(u[:, 0:D_MLP], W1_ref[0:D_MLP, :],
                preferred_element_type=jnp.float32)
    x = x + jnp.dot(i[:, 0:D_MLP], W1_ref[D_MLP:2 * D_MLP, :],
                    preferred_element_type=jnp.float32)
    x = jax.nn.relu(x + b1_ref[...])
    x = jnp.dot(x, W2_ref[...], preferred_element_type=jnp.float32)
    x = jax.nn.relu(x + b2_ref[...])
    wg = Wo_ref[0:D_GMF, 0]
    wm = Wo_ref[D_GMF:2 * D_GMF, 0]
    logit = (jnp.sum(gmf * wg[None, :], axis=-1)
             + jnp.sum(x * wm[None, :], axis=-1)
             + bo_ref[...])
    out_ref[...] = jax.nn.sigmoid(logit)


def _tc_dense(u_pack, i_pack, W1, b1, W2, b2, Wo, bo):
    grid = (B // BLK,)
    blk = pl.BlockSpec((BLK * 128,), lambda i: (i,))
    full = lambda s: pl.BlockSpec(s, lambda i: tuple(0 for _ in s))
    return pl.pallas_call(
        _dense_body,
        grid=grid,
        in_specs=[
            blk, blk,
            full(W1.shape), full(b1.shape), full(W2.shape), full(b2.shape),
            full(Wo.shape), full(bo.shape),
        ],
        out_specs=pl.BlockSpec((BLK,), lambda i: (i,)),
        out_shape=jax.ShapeDtypeStruct((B,), jnp.float32),
    )(u_pack, i_pack, W1, b1, W2, b2, Wo, bo)


def kernel(user_emb, item_emb, user_emb_mlp, item_emb_mlp,
           W1, b1, W2, b2, Wo, bo, user_ids, movie_ids):
    uid2d = user_ids.astype(jnp.int32).reshape(B // CHUNK, CHUNK)
    mid2d = movie_ids.astype(jnp.int32).reshape(B // CHUNK, CHUNK)
    u_pack = _sc_gather_side(uid2d, user_emb, user_emb_mlp)
    i_pack = _sc_gather_side(mid2d, item_emb, item_emb_mlp)
    return _tc_dense(u_pack.reshape(-1), i_pack.reshape(-1),
                     W1, b1, W2, b2, Wo, bo)
